# baseline (speedup 1.0000x reference)
import functools

import jax
import jax.numpy as jnp
from jax import lax
from jax.experimental import pallas as pl
from jax.experimental.pallas import tpu as pltpu
from jax.experimental.pallas import tpu_sc as plsc

_EPS = 1e-5
_K = 20
_BIG = 3.0e38
_IDX_PAD = 32


def _knn_a_body(x_ref, xt_ref, wa_ref, b_ref, idx_ref, a_ref, *, n, k):
    x = x_ref[...]
    xt = xt_ref[...]
    sq_all = jnp.sum(xt * xt, axis=0, keepdims=True)
    sq_row = jnp.sum(x * x, axis=1, keepdims=True)
    d = sq_row + sq_all - 2.0 * jnp.dot(x, xt, preferred_element_type=jnp.float32)
    col = lax.broadcasted_iota(jnp.int32, d.shape, 1)
    r = d.shape[0]
    lane = lax.broadcasted_iota(jnp.int32, (r, _IDX_PAD), 1)
    idx_acc = jnp.zeros((r, _IDX_PAD), jnp.int32)
    m = jnp.min(d, axis=1, keepdims=True)
    for j in range(k):
        c = jnp.min(jnp.where(d == m, col, n), axis=1, keepdims=True)
        idx_acc = jnp.where(lane == j, c, idx_acc)
        if j < k - 1:
            d = jnp.where(col == c, _BIG, d)
            m = jnp.min(d, axis=1, keepdims=True)
    idx_ref[...] = idx_acc
    a_ref[...] = jnp.dot(x, wa_ref[...], preferred_element_type=jnp.float32) + b_ref[...]


def _knn_a(x, wa, b, blk=256):
    n, dch = x.shape
    f = wa.shape[1]
    xt = x.T
    return pl.pallas_call(
        functools.partial(_knn_a_body, n=n, k=_K),
        grid=(n // blk,),
        in_specs=[
            pl.BlockSpec((blk, dch), lambda i: (i, 0)),
            pl.BlockSpec((dch, n), lambda i: (0, 0)),
            pl.BlockSpec((dch, f), lambda i: (0, 0)),
            pl.BlockSpec((1, f), lambda i: (0, 0)),
        ],
        out_specs=[
            pl.BlockSpec((blk, _IDX_PAD), lambda i: (i, 0)),
            pl.BlockSpec((blk, f), lambda i: (i, 0)),
        ],
        out_shape=[
            jax.ShapeDtypeStruct((n, _IDX_PAD), jnp.int32),
            jax.ShapeDtypeStruct((n, f), jnp.float32),
        ],
    )(x, xt, wa, b.reshape(1, f))


def _sc_gather(table, idx):
    nw, nchunk, cw = idx.shape
    n, f = table.shape
    rows_out = nw * nchunk * cw
    per_w = nchunk * cw
    info = plsc.get_sparse_core_info()
    nc = info.num_cores
    mesh = plsc.VectorSubcoreMesh(core_axis_name="c", subcore_axis_name="s")

    @functools.partial(
        pl.kernel,
        out_type=jax.ShapeDtypeStruct((rows_out, f), jnp.float32),
        mesh=mesh,
        scratch_types=[
            pltpu.VMEM((nchunk, cw), jnp.int32),
            pltpu.VMEM((cw, f), jnp.float32),
            pltpu.SemaphoreType.DMA,
        ],
    )
    def gk(table_hbm, idx_hbm, out_hbm, idx_v, rows_v, sem):
        wid = lax.axis_index("s") * nc + lax.axis_index("c")
        pltpu.sync_copy(idx_hbm.at[wid], idx_v)

        def body(j, carry):
            pltpu.async_copy(table_hbm.at[idx_v.at[j]], rows_v, sem).wait()
            pltpu.sync_copy(rows_v, out_hbm.at[pl.ds(wid * per_w + j * cw, cw)])
            return carry

        lax.fori_loop(0, nchunk, body, 0)

    return gk(table, idx)


def _edge_mlp3_body(a_ref, g_ref, x_ref, wb_ref, w1_ref, b1_ref, w2_ref, b2_ref,
                    s0_ref, t0_ref, s1_ref, t1_ref, s2_ref, t2_ref,
                    x1_ref, *, k):
    a = a_ref[...]
    xi = x_ref[...]
    acc = None
    for s in range(k):
        e = g_ref[s] - xi
        h = a + jnp.dot(e, wb_ref[...], preferred_element_type=jnp.float32)
        h = jnp.maximum(h, 0.0) * s0_ref[...] + t0_ref[...]
        h = jnp.maximum(
            jnp.dot(h, w1_ref[...], preferred_element_type=jnp.float32) + b1_ref[...],
            0.0) * s1_ref[...] + t1_ref[...]
        h = jnp.maximum(
            jnp.dot(h, w2_ref[...], preferred_element_type=jnp.float32) + b2_ref[...],
            0.0) * s2_ref[...] + t2_ref[...]
        acc = h if acc is None else jnp.maximum(acc, h)
    x1_ref[...] = jnp.concatenate(
        [acc, jnp.zeros((acc.shape[0], 128 - acc.shape[1]), acc.dtype)], axis=1)


def _edge_mlp3(a, g, x, wb, w1, b1, w2, b2, s0, t0, s1, t1, s2, t2, blk=256):
    n, f = a.shape
    dch = x.shape[1]
    vec = lambda v: v.reshape(1, f)
    full = lambda i: (0, 0)
    return pl.pallas_call(
        functools.partial(_edge_mlp3_body, k=_K),
        grid=(n // blk,),
        in_specs=[
            pl.BlockSpec((blk, f), lambda i: (i, 0)),
            pl.BlockSpec((_K, blk, dch), lambda i: (0, i, 0)),
            pl.BlockSpec((blk, dch), lambda i: (i, 0)),
            pl.BlockSpec((dch, f), full),
            pl.BlockSpec((f, f), full),
            pl.BlockSpec((1, f), full),
            pl.BlockSpec((f, f), full),
            pl.BlockSpec((1, f), full),
        ] + [pl.BlockSpec((1, f), full)] * 6,
        out_specs=pl.BlockSpec((blk, 128), lambda i: (i, 0)),
        out_shape=jax.ShapeDtypeStruct((n, 128), jnp.float32),
    )(a, g, x, wb, w1, vec(b1), w2, vec(b2),
      vec(s0), vec(t0), vec(s1), vec(t1), vec(s2), vec(t2))


def _edge_final_body(a2_ref, g2_ref, x1_ref, wb2_ref, s0_ref, t0_ref,
                     w3a_ref, w3b_ref, b3_ref, s3_ref, t3_ref,
                     out_ref, *, k):
    a2 = a2_ref[...]
    x1 = x1_ref[...]
    acc = None
    for s in range(k):
        e = g2_ref[s] - x1
        h = a2 + jnp.dot(e, wb2_ref[...], preferred_element_type=jnp.float32)
        h = jnp.maximum(h, 0.0) * s0_ref[...] + t0_ref[...]
        acc = h if acc is None else jnp.maximum(acc, h)
    o = (jnp.dot(x1, w3a_ref[...], preferred_element_type=jnp.float32)
         + jnp.dot(acc, w3b_ref[...], preferred_element_type=jnp.float32)
         + b3_ref[...])
    out_ref[...] = jnp.maximum(o, 0.0) * s3_ref[...] + t3_ref[...]


def _edge_final(a2, g2, x1, wb2, s0, t0, w3a, w3b, b3, s3, t3, blk=256):
    n, f2 = a2.shape
    dch = x1.shape[1]
    vec2 = lambda v: v.reshape(1, f2)
    full = lambda i: (0, 0)
    return pl.pallas_call(
        functools.partial(_edge_final_body, k=_K),
        grid=(n // blk,),
        in_specs=[
            pl.BlockSpec((blk, f2), lambda i: (i, 0)),
            pl.BlockSpec((_K, blk, dch), lambda i: (0, i, 0)),
            pl.BlockSpec((blk, dch), lambda i: (i, 0)),
            pl.BlockSpec((dch, f2), full),
            pl.BlockSpec((1, f2), full),
            pl.BlockSpec((1, f2), full),
            pl.BlockSpec((dch, f2), full),
            pl.BlockSpec((f2, f2), full),
            pl.BlockSpec((1, f2), full),
            pl.BlockSpec((1, f2), full),
            pl.BlockSpec((1, f2), full),
        ],
        out_specs=pl.BlockSpec((blk, f2), lambda i: (i, 0)),
        out_shape=jax.ShapeDtypeStruct((n, f2), jnp.float32),
    )(a2, g2, x1, wb2, vec2(s0), vec2(t0), w3a, w3b, vec2(b3), vec2(s3), vec2(t3))


def _idx_for_sc(idx_padded, n):
    flat = jnp.transpose(idx_padded[:, :_K]).reshape(-1)
    return flat.reshape(32, (_K * n) // (32 * 128), 128)


def _pad_rows(w, rows):
    return jnp.pad(w, ((0, rows - w.shape[0]), (0, 0)))


def kernel(cell_boxes, fusion_feat,
           W1_0, b1_0, g1_0, be1_0, W1_1, b1_1, g1_1, be1_1,
           W1_2, b1_2, g1_2, be1_2, W2_0, b2_0, g2_0, be2_0,
           W3_0, b3_0, g3_0, be3_0):
    del cell_boxes
    n, d = fusion_feat.shape
    inv = 1.0 / jnp.sqrt(jnp.float32(1.0 + _EPS))

    idx1, a1 = _knn_a(fusion_feat, W1_0[:d], b1_0)

    g1 = _sc_gather(fusion_feat, _idx_for_sc(idx1, n)).reshape(_K, n, -1)
    x1p = _edge_mlp3(a1, g1, fusion_feat, W1_0[d:], W1_1, b1_1, W1_2, b1_2,
                     g1_0 * inv, be1_0, g1_1 * inv, be1_1, g1_2 * inv, be1_2)

    f1 = W1_2.shape[1]
    idx2, a2 = _knn_a(x1p, _pad_rows(W2_0[:f1], 128), b2_0)

    g2 = _sc_gather(x1p, _idx_for_sc(idx2, n)).reshape(_K, n, -1)
    out = _edge_final(a2, g2, x1p, _pad_rows(W2_0[f1:], 128), g2_0 * inv, be2_0,
                      _pad_rows(W3_0[:f1], 128), W3_0[f1:], b3_0,
                      g3_0 * inv, be3_0)
    return out


# speedup vs baseline: 1.1400x; 1.1400x over previous
import functools

import jax
import jax.numpy as jnp
from jax import lax
from jax.experimental import pallas as pl
from jax.experimental.pallas import tpu as pltpu
from jax.experimental.pallas import tpu_sc as plsc

_EPS = 1e-5
_K = 20
_BIG = 3.0e38
_IDX_PAD = 32


def _topk_iter(vals, cols, sentinel, k):
    r = vals.shape[0]
    lane = lax.broadcasted_iota(jnp.int32, (r, _IDX_PAD), 1)
    idx_acc = jnp.zeros((r, _IDX_PAD), jnp.int32)
    m = jnp.min(vals, axis=1, keepdims=True)
    for j in range(k):
        c = jnp.min(jnp.where(vals == m, cols, sentinel), axis=1, keepdims=True)
        idx_acc = jnp.where(lane == j, c, idx_acc)
        if j < k - 1:
            vals = jnp.where(cols == c, _BIG, vals)
            m = jnp.min(vals, axis=1, keepdims=True)
    return idx_acc, m


def _knn_a_body(x_ref, xt_ref, wa_ref, b_ref, idx_ref, a_ref, *, n, k):
    x = x_ref[...]
    xt = xt_ref[...]
    sq_all = jnp.sum(xt * xt, axis=0, keepdims=True)
    sq_row = jnp.sum(x * x, axis=1, keepdims=True)
    d = sq_row + sq_all - 2.0 * jnp.dot(x, xt, preferred_element_type=jnp.float32)
    col = lax.broadcasted_iota(jnp.int32, d.shape, 1)
    r = d.shape[0]
    ng = n // 128
    m1 = m2 = m3 = m4 = jnp.full((r, 128), _BIG, jnp.float32)
    a1 = a2 = a3 = a4 = jnp.zeros((r, 128), jnp.int32)
    for g in range(ng):
        s = d[:, g * 128:(g + 1) * 128]
        p1 = s < m1
        p2 = s < m2
        p3 = s < m3
        p4 = s < m4
        m4 = jnp.where(p4, jnp.where(p3, m3, s), m4)
        a4 = jnp.where(p4, jnp.where(p3, a3, g), a4)
        m3 = jnp.where(p3, jnp.where(p2, m2, s), m3)
        a3 = jnp.where(p3, jnp.where(p2, a2, g), a3)
        m2 = jnp.where(p2, jnp.where(p1, m1, s), m2)
        a2 = jnp.where(p2, jnp.where(p1, a1, g), a2)
        m1 = jnp.where(p1, s, m1)
        a1 = jnp.where(p1, g, a1)
    lane128 = lax.broadcasted_iota(jnp.int32, (r, 128), 1)
    vals = jnp.concatenate([m1, m2, m3, m4], axis=1)
    cols = jnp.concatenate([a1 * 128 + lane128, a2 * 128 + lane128,
                            a3 * 128 + lane128, a4 * 128 + lane128], axis=1)
    idx_fast, t20 = _topk_iter(vals, cols, n, k)
    bad = jnp.max(jnp.where(m4 <= t20, 1, 0))
    idx_ref[...] = lax.cond(
        bad > 0,
        lambda: _topk_iter(d, col, n, k)[0],
        lambda: idx_fast)
    a_ref[...] = jnp.dot(x, wa_ref[...], preferred_element_type=jnp.float32) + b_ref[...]


def _knn_a(x, wa, b, blk=256):
    n, dch = x.shape
    f = wa.shape[1]
    xt = x.T
    return pl.pallas_call(
        functools.partial(_knn_a_body, n=n, k=_K),
        grid=(n // blk,),
        in_specs=[
            pl.BlockSpec((blk, dch), lambda i: (i, 0)),
            pl.BlockSpec((dch, n), lambda i: (0, 0)),
            pl.BlockSpec((dch, f), lambda i: (0, 0)),
            pl.BlockSpec((1, f), lambda i: (0, 0)),
        ],
        out_specs=[
            pl.BlockSpec((blk, _IDX_PAD), lambda i: (i, 0)),
            pl.BlockSpec((blk, f), lambda i: (i, 0)),
        ],
        out_shape=[
            jax.ShapeDtypeStruct((n, _IDX_PAD), jnp.int32),
            jax.ShapeDtypeStruct((n, f), jnp.float32),
        ],
    )(x, xt, wa, b.reshape(1, f))


def _sc_gather(table, idx):
    nw, nchunk, cw = idx.shape
    n, f = table.shape
    rows_out = nw * nchunk * cw
    per_w = nchunk * cw
    info = plsc.get_sparse_core_info()
    nc = info.num_cores
    mesh = plsc.VectorSubcoreMesh(core_axis_name="c", subcore_axis_name="s")

    @functools.partial(
        pl.kernel,
        out_type=jax.ShapeDtypeStruct((rows_out, f), jnp.float32),
        mesh=mesh,
        scratch_types=[
            pltpu.VMEM((nchunk, cw), jnp.int32),
            pltpu.VMEM((cw, f), jnp.float32),
            pltpu.SemaphoreType.DMA,
        ],
    )
    def gk(table_hbm, idx_hbm, out_hbm, idx_v, rows_v, sem):
        wid = lax.axis_index("s") * nc + lax.axis_index("c")
        pltpu.sync_copy(idx_hbm.at[wid], idx_v)

        def body(j, carry):
            pltpu.async_copy(table_hbm.at[idx_v.at[j]], rows_v, sem).wait()
            pltpu.sync_copy(rows_v, out_hbm.at[pl.ds(wid * per_w + j * cw, cw)])
            return carry

        lax.fori_loop(0, nchunk, body, 0)

    return gk(table, idx)


def _edge_mlp3_body(a_ref, g_ref, x_ref, wb_ref, w1_ref, b1_ref, w2_ref, b2_ref,
                    s0_ref, t0_ref, s1_ref, t1_ref, s2_ref, t2_ref,
                    x1_ref, *, k):
    a = a_ref[...]
    xi = x_ref[...]
    acc = None
    for s in range(k):
        e = g_ref[s] - xi
        h = a + jnp.dot(e, wb_ref[...], preferred_element_type=jnp.float32)
        h = jnp.maximum(h, 0.0) * s0_ref[...] + t0_ref[...]
        h = jnp.maximum(
            jnp.dot(h, w1_ref[...], preferred_element_type=jnp.float32) + b1_ref[...],
            0.0) * s1_ref[...] + t1_ref[...]
        h = jnp.maximum(
            jnp.dot(h, w2_ref[...], preferred_element_type=jnp.float32) + b2_ref[...],
            0.0) * s2_ref[...] + t2_ref[...]
        acc = h if acc is None else jnp.maximum(acc, h)
    x1_ref[...] = jnp.concatenate(
        [acc, jnp.zeros((acc.shape[0], 128 - acc.shape[1]), acc.dtype)], axis=1)


def _edge_mlp3(a, g, x, wb, w1, b1, w2, b2, s0, t0, s1, t1, s2, t2, blk=256):
    n, f = a.shape
    dch = x.shape[1]
    vec = lambda v: v.reshape(1, f)
    full = lambda i: (0, 0)
    return pl.pallas_call(
        functools.partial(_edge_mlp3_body, k=_K),
        grid=(n // blk,),
        in_specs=[
            pl.BlockSpec((blk, f), lambda i: (i, 0)),
            pl.BlockSpec((_K, blk, dch), lambda i: (0, i, 0)),
            pl.BlockSpec((blk, dch), lambda i: (i, 0)),
            pl.BlockSpec((dch, f), full),
            pl.BlockSpec((f, f), full),
            pl.BlockSpec((1, f), full),
            pl.BlockSpec((f, f), full),
            pl.BlockSpec((1, f), full),
        ] + [pl.BlockSpec((1, f), full)] * 6,
        out_specs=pl.BlockSpec((blk, 128), lambda i: (i, 0)),
        out_shape=jax.ShapeDtypeStruct((n, 128), jnp.float32),
    )(a, g, x, wb, w1, vec(b1), w2, vec(b2),
      vec(s0), vec(t0), vec(s1), vec(t1), vec(s2), vec(t2))


def _edge_final_body(a2_ref, g2_ref, x1_ref, wb2_ref, s0_ref, t0_ref,
                     w3a_ref, w3b_ref, b3_ref, s3_ref, t3_ref,
                     out_ref, *, k):
    a2 = a2_ref[...]
    x1 = x1_ref[...]
    acc = None
    for s in range(k):
        e = g2_ref[s] - x1
        h = a2 + jnp.dot(e, wb2_ref[...], preferred_element_type=jnp.float32)
        h = jnp.maximum(h, 0.0) * s0_ref[...] + t0_ref[...]
        acc = h if acc is None else jnp.maximum(acc, h)
    o = (jnp.dot(x1, w3a_ref[...], preferred_element_type=jnp.float32)
         + jnp.dot(acc, w3b_ref[...], preferred_element_type=jnp.float32)
         + b3_ref[...])
    out_ref[...] = jnp.maximum(o, 0.0) * s3_ref[...] + t3_ref[...]


def _edge_final(a2, g2, x1, wb2, s0, t0, w3a, w3b, b3, s3, t3, blk=256):
    n, f2 = a2.shape
    dch = x1.shape[1]
    vec2 = lambda v: v.reshape(1, f2)
    full = lambda i: (0, 0)
    return pl.pallas_call(
        functools.partial(_edge_final_body, k=_K),
        grid=(n // blk,),
        in_specs=[
            pl.BlockSpec((blk, f2), lambda i: (i, 0)),
            pl.BlockSpec((_K, blk, dch), lambda i: (0, i, 0)),
            pl.BlockSpec((blk, dch), lambda i: (i, 0)),
            pl.BlockSpec((dch, f2), full),
            pl.BlockSpec((1, f2), full),
            pl.BlockSpec((1, f2), full),
            pl.BlockSpec((dch, f2), full),
            pl.BlockSpec((f2, f2), full),
            pl.BlockSpec((1, f2), full),
            pl.BlockSpec((1, f2), full),
            pl.BlockSpec((1, f2), full),
        ],
        out_specs=pl.BlockSpec((blk, f2), lambda i: (i, 0)),
        out_shape=jax.ShapeDtypeStruct((n, f2), jnp.float32),
    )(a2, g2, x1, wb2, vec2(s0), vec2(t0), w3a, w3b, vec2(b3), vec2(s3), vec2(t3))


def _idx_for_sc(idx_padded, n):
    flat = jnp.transpose(idx_padded[:, :_K]).reshape(-1)
    return flat.reshape(32, (_K * n) // (32 * 128), 128)


def _pad_rows(w, rows):
    return jnp.pad(w, ((0, rows - w.shape[0]), (0, 0)))


def kernel(cell_boxes, fusion_feat,
           W1_0, b1_0, g1_0, be1_0, W1_1, b1_1, g1_1, be1_1,
           W1_2, b1_2, g1_2, be1_2, W2_0, b2_0, g2_0, be2_0,
           W3_0, b3_0, g3_0, be3_0):
    del cell_boxes
    n, d = fusion_feat.shape
    inv = 1.0 / jnp.sqrt(jnp.float32(1.0 + _EPS))

    idx1, a1 = _knn_a(fusion_feat, W1_0[:d], b1_0)

    g1 = _sc_gather(fusion_feat, _idx_for_sc(idx1, n)).reshape(_K, n, -1)
    x1p = _edge_mlp3(a1, g1, fusion_feat, W1_0[d:], W1_1, b1_1, W1_2, b1_2,
                     g1_0 * inv, be1_0, g1_1 * inv, be1_1, g1_2 * inv, be1_2)

    f1 = W1_2.shape[1]
    idx2, a2 = _knn_a(x1p, _pad_rows(W2_0[:f1], 128), b2_0)

    g2 = _sc_gather(x1p, _idx_for_sc(idx2, n)).reshape(_K, n, -1)
    out = _edge_final(a2, g2, x1p, _pad_rows(W2_0[f1:], 128), g2_0 * inv, be2_0,
                      _pad_rows(W3_0[:f1], 128), W3_0[f1:], b3_0,
                      g3_0 * inv, be3_0)
    return out


# speedup vs baseline: 1.6370x; 1.4360x over previous
import functools

import jax
import jax.numpy as jnp
from jax import lax
from jax.experimental import pallas as pl
from jax.experimental.pallas import tpu as pltpu
from jax.experimental.pallas import tpu_sc as plsc

_EPS = 1e-5
_K = 20
_BIG = 3.0e38
_IDX_PAD = 32


def _topk_iter(vals, cols, sentinel, k):
    r = vals.shape[0]
    lane = lax.broadcasted_iota(jnp.int32, (r, _IDX_PAD), 1)
    idx_acc = jnp.zeros((r, _IDX_PAD), jnp.int32)
    m = jnp.min(vals, axis=1, keepdims=True)
    for j in range(k):
        c = jnp.min(jnp.where(vals == m, cols, sentinel), axis=1, keepdims=True)
        idx_acc = jnp.where(lane == j, c, idx_acc)
        if j < k - 1:
            vals = jnp.where(cols == c, _BIG, vals)
            m = jnp.min(vals, axis=1, keepdims=True)
    return idx_acc, m


def _knn_a_body(x_ref, xt_ref, wa_ref, b_ref, idx_ref, a_ref, *, n, k):
    x = x_ref[...]
    xt = xt_ref[...]
    sq_all = jnp.sum(xt * xt, axis=0, keepdims=True)
    sq_row = jnp.sum(x * x, axis=1, keepdims=True)
    d = sq_row + sq_all - 2.0 * jnp.dot(x, xt, preferred_element_type=jnp.float32)
    col = lax.broadcasted_iota(jnp.int32, d.shape, 1)
    r = d.shape[0]
    ng = n // 128
    nb = 5
    mv = [jnp.full((r, 128), _BIG, jnp.float32) for _ in range(nb)]
    av = [jnp.zeros((r, 128), jnp.int32) for _ in range(nb)]
    for g in range(ng):
        s = d[:, g * 128:(g + 1) * 128]
        p = [s < mv[i] for i in range(nb)]
        for i in range(nb - 1, 0, -1):
            mv[i] = jnp.where(p[i], jnp.where(p[i - 1], mv[i - 1], s), mv[i])
            av[i] = jnp.where(p[i], jnp.where(p[i - 1], av[i - 1], g), av[i])
        mv[0] = jnp.where(p[0], s, mv[0])
        av[0] = jnp.where(p[0], g, av[0])
    lane128 = lax.broadcasted_iota(jnp.int32, (r, 128), 1)
    vals = jnp.concatenate(mv, axis=1)
    cols = jnp.concatenate([a * 128 + lane128 for a in av], axis=1)
    idx_fast, t20 = _topk_iter(vals, cols, n, k)
    bad = jnp.max(jnp.where(mv[nb - 1] <= t20, 1, 0))
    idx_ref[...] = lax.cond(
        bad > 0,
        lambda: _topk_iter(d, col, n, k)[0],
        lambda: idx_fast)
    a_ref[...] = jnp.dot(x, wa_ref[...], preferred_element_type=jnp.float32) + b_ref[...]


def _knn_a(x, wa, b, blk=256):
    n, dch = x.shape
    f = wa.shape[1]
    xt = x.T
    return pl.pallas_call(
        functools.partial(_knn_a_body, n=n, k=_K),
        grid=(n // blk,),
        in_specs=[
            pl.BlockSpec((blk, dch), lambda i: (i, 0)),
            pl.BlockSpec((dch, n), lambda i: (0, 0)),
            pl.BlockSpec((dch, f), lambda i: (0, 0)),
            pl.BlockSpec((1, f), lambda i: (0, 0)),
        ],
        out_specs=[
            pl.BlockSpec((blk, _IDX_PAD), lambda i: (i, 0)),
            pl.BlockSpec((blk, f), lambda i: (i, 0)),
        ],
        out_shape=[
            jax.ShapeDtypeStruct((n, _IDX_PAD), jnp.int32),
            jax.ShapeDtypeStruct((n, f), jnp.float32),
        ],
    )(x, xt, wa, b.reshape(1, f))


def _sc_gather(table, idx):
    nw, nchunk, cw = idx.shape
    n, f = table.shape
    rows_out = nw * nchunk * cw
    per_w = nchunk * cw
    info = plsc.get_sparse_core_info()
    nc = info.num_cores
    mesh = plsc.VectorSubcoreMesh(core_axis_name="c", subcore_axis_name="s")

    @functools.partial(
        pl.kernel,
        out_type=jax.ShapeDtypeStruct((rows_out, f), jnp.float32),
        mesh=mesh,
        scratch_types=[
            pltpu.VMEM((nchunk, cw), jnp.int32),
            pltpu.VMEM((cw, f), jnp.float32),
            pltpu.SemaphoreType.DMA,
        ],
    )
    def gk(table_hbm, idx_hbm, out_hbm, idx_v, rows_v, sem):
        wid = lax.axis_index("s") * nc + lax.axis_index("c")
        pltpu.sync_copy(idx_hbm.at[wid], idx_v)

        def body(j, carry):
            pltpu.async_copy(table_hbm.at[idx_v.at[j]], rows_v, sem).wait()
            pltpu.sync_copy(rows_v, out_hbm.at[pl.ds(wid * per_w + j * cw, cw)])
            return carry

        lax.fori_loop(0, nchunk, body, 0)

    return gk(table, idx)


def _edge_mlp3_body(a_ref, g_ref, x_ref, wb_ref, w1_ref, b1_ref, w2_ref, b2_ref,
                    s0_ref, t0_ref, s1_ref, t1_ref, s2_ref, t2_ref,
                    x1_ref, *, k):
    a = a_ref[...]
    xi = x_ref[...]
    acc = None
    for s in range(k):
        e = g_ref[s] - xi
        h = a + jnp.dot(e, wb_ref[...], preferred_element_type=jnp.float32)
        h = jnp.maximum(h, 0.0) * s0_ref[...] + t0_ref[...]
        h = jnp.maximum(
            jnp.dot(h, w1_ref[...], preferred_element_type=jnp.float32) + b1_ref[...],
            0.0) * s1_ref[...] + t1_ref[...]
        h = jnp.maximum(
            jnp.dot(h, w2_ref[...], preferred_element_type=jnp.float32) + b2_ref[...],
            0.0) * s2_ref[...] + t2_ref[...]
        acc = h if acc is None else jnp.maximum(acc, h)
    x1_ref[...] = jnp.concatenate(
        [acc, jnp.zeros((acc.shape[0], 128 - acc.shape[1]), acc.dtype)], axis=1)


def _edge_mlp3(a, g, x, wb, w1, b1, w2, b2, s0, t0, s1, t1, s2, t2, blk=256):
    n, f = a.shape
    dch = x.shape[1]
    vec = lambda v: v.reshape(1, f)
    full = lambda i: (0, 0)
    return pl.pallas_call(
        functools.partial(_edge_mlp3_body, k=_K),
        grid=(n // blk,),
        in_specs=[
            pl.BlockSpec((blk, f), lambda i: (i, 0)),
            pl.BlockSpec((_K, blk, dch), lambda i: (0, i, 0)),
            pl.BlockSpec((blk, dch), lambda i: (i, 0)),
            pl.BlockSpec((dch, f), full),
            pl.BlockSpec((f, f), full),
            pl.BlockSpec((1, f), full),
            pl.BlockSpec((f, f), full),
            pl.BlockSpec((1, f), full),
        ] + [pl.BlockSpec((1, f), full)] * 6,
        out_specs=pl.BlockSpec((blk, 128), lambda i: (i, 0)),
        out_shape=jax.ShapeDtypeStruct((n, 128), jnp.float32),
    )(a, g, x, wb, w1, vec(b1), w2, vec(b2),
      vec(s0), vec(t0), vec(s1), vec(t1), vec(s2), vec(t2))


def _edge_final_body(a2_ref, g2_ref, x1_ref, wb2_ref, s0_ref, t0_ref,
                     w3a_ref, w3b_ref, b3_ref, s3_ref, t3_ref,
                     out_ref, *, k):
    a2 = a2_ref[...]
    x1 = x1_ref[...]
    acc = None
    for s in range(k):
        e = g2_ref[s] - x1
        h = a2 + jnp.dot(e, wb2_ref[...], preferred_element_type=jnp.float32)
        h = jnp.maximum(h, 0.0) * s0_ref[...] + t0_ref[...]
        acc = h if acc is None else jnp.maximum(acc, h)
    o = (jnp.dot(x1, w3a_ref[...], preferred_element_type=jnp.float32)
         + jnp.dot(acc, w3b_ref[...], preferred_element_type=jnp.float32)
         + b3_ref[...])
    out_ref[...] = jnp.maximum(o, 0.0) * s3_ref[...] + t3_ref[...]


def _edge_final(a2, g2, x1, wb2, s0, t0, w3a, w3b, b3, s3, t3, blk=256):
    n, f2 = a2.shape
    dch = x1.shape[1]
    vec2 = lambda v: v.reshape(1, f2)
    full = lambda i: (0, 0)
    return pl.pallas_call(
        functools.partial(_edge_final_body, k=_K),
        grid=(n // blk,),
        in_specs=[
            pl.BlockSpec((blk, f2), lambda i: (i, 0)),
            pl.BlockSpec((_K, blk, dch), lambda i: (0, i, 0)),
            pl.BlockSpec((blk, dch), lambda i: (i, 0)),
            pl.BlockSpec((dch, f2), full),
            pl.BlockSpec((1, f2), full),
            pl.BlockSpec((1, f2), full),
            pl.BlockSpec((dch, f2), full),
            pl.BlockSpec((f2, f2), full),
            pl.BlockSpec((1, f2), full),
            pl.BlockSpec((1, f2), full),
            pl.BlockSpec((1, f2), full),
        ],
        out_specs=pl.BlockSpec((blk, f2), lambda i: (i, 0)),
        out_shape=jax.ShapeDtypeStruct((n, f2), jnp.float32),
    )(a2, g2, x1, wb2, vec2(s0), vec2(t0), w3a, w3b, vec2(b3), vec2(s3), vec2(t3))


def _idx_for_sc(idx_padded, n):
    flat = jnp.transpose(idx_padded[:, :_K]).reshape(-1)
    return flat.reshape(32, (_K * n) // (32 * 128), 128)


def _pad_rows(w, rows):
    return jnp.pad(w, ((0, rows - w.shape[0]), (0, 0)))


def kernel(cell_boxes, fusion_feat,
           W1_0, b1_0, g1_0, be1_0, W1_1, b1_1, g1_1, be1_1,
           W1_2, b1_2, g1_2, be1_2, W2_0, b2_0, g2_0, be2_0,
           W3_0, b3_0, g3_0, be3_0):
    del cell_boxes
    n, d = fusion_feat.shape
    inv = 1.0 / jnp.sqrt(jnp.float32(1.0 + _EPS))

    idx1, a1 = _knn_a(fusion_feat, W1_0[:d], b1_0)

    g1 = _sc_gather(fusion_feat, _idx_for_sc(idx1, n)).reshape(_K, n, -1)
    x1p = _edge_mlp3(a1, g1, fusion_feat, W1_0[d:], W1_1, b1_1, W1_2, b1_2,
                     g1_0 * inv, be1_0, g1_1 * inv, be1_1, g1_2 * inv, be1_2)

    f1 = W1_2.shape[1]
    idx2, a2 = _knn_a(x1p, _pad_rows(W2_0[:f1], 128), b2_0)

    g2 = _sc_gather(x1p, _idx_for_sc(idx2, n)).reshape(_K, n, -1)
    out = _edge_final(a2, g2, x1p, _pad_rows(W2_0[f1:], 128), g2_0 * inv, be2_0,
                      _pad_rows(W3_0[:f1], 128), W3_0[f1:], b3_0,
                      g3_0 * inv, be3_0)
    return out


# speedup vs baseline: 1.6446x; 1.0046x over previous
import functools

import jax
import jax.numpy as jnp
from jax import lax
from jax.experimental import pallas as pl
from jax.experimental.pallas import tpu as pltpu
from jax.experimental.pallas import tpu_sc as plsc

_EPS = 1e-5
_K = 20
_BIG = 3.0e38
_IDX_PAD = 32


def _topk_iter(vals, cols, sentinel, k):
    r = vals.shape[0]
    lane = lax.broadcasted_iota(jnp.int32, (r, _IDX_PAD), 1)
    idx_acc = jnp.zeros((r, _IDX_PAD), jnp.int32)
    m = jnp.min(vals, axis=1, keepdims=True)
    for j in range(k):
        c = jnp.min(jnp.where(vals == m, cols, sentinel), axis=1, keepdims=True)
        idx_acc = jnp.where(lane == j, c, idx_acc)
        if j < k - 1:
            vals = jnp.where(cols == c, _BIG, vals)
            m = jnp.min(vals, axis=1, keepdims=True)
    return idx_acc, m


def _knn_a_body(x_ref, xt_ref, wa_ref, b_ref, idx_ref, a_ref, *, n, k):
    x = x_ref[...]
    xt = xt_ref[...]
    sq_all = jnp.sum(xt * xt, axis=0, keepdims=True)
    sq_row = jnp.sum(x * x, axis=1, keepdims=True)
    d = sq_row + sq_all - 2.0 * jnp.dot(x, xt, preferred_element_type=jnp.float32)
    col = lax.broadcasted_iota(jnp.int32, d.shape, 1)
    r = d.shape[0]
    ng = n // 128
    nb = 5
    mv = [jnp.full((r, 128), _BIG, jnp.float32) for _ in range(nb)]
    av = [jnp.zeros((r, 128), jnp.int32) for _ in range(nb)]
    for g in range(ng):
        s = d[:, g * 128:(g + 1) * 128]
        p = [s < mv[i] for i in range(nb)]
        for i in range(nb - 1, 0, -1):
            mv[i] = jnp.where(p[i], jnp.where(p[i - 1], mv[i - 1], s), mv[i])
            av[i] = jnp.where(p[i], jnp.where(p[i - 1], av[i - 1], g), av[i])
        mv[0] = jnp.where(p[0], s, mv[0])
        av[0] = jnp.where(p[0], g, av[0])
    lane128 = lax.broadcasted_iota(jnp.int32, (r, 128), 1)
    vals = jnp.concatenate(mv, axis=1)
    cols = jnp.concatenate([a * 128 + lane128 for a in av], axis=1)
    idx_fast, t20 = _topk_iter(vals, cols, n, k)
    bad = jnp.max(jnp.where(mv[nb - 1] <= t20, 1, 0))
    idx_ref[...] = lax.cond(
        bad > 0,
        lambda: _topk_iter(d, col, n, k)[0],
        lambda: idx_fast)
    a_ref[...] = jnp.dot(x, wa_ref[...], preferred_element_type=jnp.float32) + b_ref[...]


def _knn_a(x, wa, b, blk=256):
    n, dch = x.shape
    f = wa.shape[1]
    xt = x.T
    return pl.pallas_call(
        functools.partial(_knn_a_body, n=n, k=_K),
        grid=(n // blk,),
        in_specs=[
            pl.BlockSpec((blk, dch), lambda i: (i, 0)),
            pl.BlockSpec((dch, n), lambda i: (0, 0)),
            pl.BlockSpec((dch, f), lambda i: (0, 0)),
            pl.BlockSpec((1, f), lambda i: (0, 0)),
        ],
        out_specs=[
            pl.BlockSpec((blk, _IDX_PAD), lambda i: (i, 0)),
            pl.BlockSpec((blk, f), lambda i: (i, 0)),
        ],
        out_shape=[
            jax.ShapeDtypeStruct((n, _IDX_PAD), jnp.int32),
            jax.ShapeDtypeStruct((n, f), jnp.float32),
        ],
    )(x, xt, wa, b.reshape(1, f))


def _sc_gather(table, idx):
    nw, nchunk, cw = idx.shape
    n, f = table.shape
    rows_out = nw * nchunk * cw
    per_w = nchunk * cw
    info = plsc.get_sparse_core_info()
    nc = info.num_cores
    mesh = plsc.VectorSubcoreMesh(core_axis_name="c", subcore_axis_name="s")

    @functools.partial(
        pl.kernel,
        out_type=jax.ShapeDtypeStruct((rows_out, f), jnp.float32),
        mesh=mesh,
        scratch_types=[
            pltpu.VMEM((nchunk, cw), jnp.int32),
            pltpu.VMEM((cw, f), jnp.float32),
            pltpu.VMEM((cw, f), jnp.float32),
            pltpu.SemaphoreType.DMA,
            pltpu.SemaphoreType.DMA,
        ],
    )
    def gk(table_hbm, idx_hbm, out_hbm, idx_v, rows0, rows1, sem0, sem1):
        wid = lax.axis_index("s") * nc + lax.axis_index("c")
        base = wid * per_w
        pltpu.sync_copy(idx_hbm.at[wid], idx_v)
        pltpu.async_copy(table_hbm.at[idx_v.at[0]], rows0, sem0)

        def body(t, carry):
            j0 = 2 * t
            j1 = 2 * t + 1
            pltpu.async_copy(table_hbm.at[idx_v.at[j1]], rows1, sem1)
            pltpu.make_async_copy(table_hbm.at[idx_v.at[j0]], rows0, sem0).wait()
            pltpu.sync_copy(rows0, out_hbm.at[pl.ds(base + j0 * cw, cw)])

            @pl.when(t < nchunk // 2 - 1)
            def _():
                pltpu.async_copy(table_hbm.at[idx_v.at[j1 + 1]], rows0, sem0)

            pltpu.make_async_copy(table_hbm.at[idx_v.at[j1]], rows1, sem1).wait()
            pltpu.sync_copy(rows1, out_hbm.at[pl.ds(base + j1 * cw, cw)])
            return carry

        lax.fori_loop(0, nchunk // 2, body, 0)

    return gk(table, idx)


def _edge_mlp3_body(a_ref, g_ref, x_ref, wb_ref, w1_ref, b1_ref, w2_ref, b2_ref,
                    s0_ref, t0_ref, s1_ref, t1_ref, s2_ref, t2_ref,
                    x1_ref, *, k):
    a = a_ref[...]
    xi = x_ref[...]
    acc = None
    for s in range(k):
        e = g_ref[s] - xi
        h = a + jnp.dot(e, wb_ref[...], preferred_element_type=jnp.float32)
        h = jnp.maximum(h, 0.0) * s0_ref[...] + t0_ref[...]
        h = jnp.maximum(
            jnp.dot(h, w1_ref[...], preferred_element_type=jnp.float32) + b1_ref[...],
            0.0) * s1_ref[...] + t1_ref[...]
        h = jnp.maximum(
            jnp.dot(h, w2_ref[...], preferred_element_type=jnp.float32) + b2_ref[...],
            0.0) * s2_ref[...] + t2_ref[...]
        acc = h if acc is None else jnp.maximum(acc, h)
    x1_ref[...] = jnp.concatenate(
        [acc, jnp.zeros((acc.shape[0], 128 - acc.shape[1]), acc.dtype)], axis=1)


def _edge_mlp3(a, g, x, wb, w1, b1, w2, b2, s0, t0, s1, t1, s2, t2, blk=256):
    n, f = a.shape
    dch = x.shape[1]
    vec = lambda v: v.reshape(1, f)
    full = lambda i: (0, 0)
    return pl.pallas_call(
        functools.partial(_edge_mlp3_body, k=_K),
        grid=(n // blk,),
        in_specs=[
            pl.BlockSpec((blk, f), lambda i: (i, 0)),
            pl.BlockSpec((_K, blk, dch), lambda i: (0, i, 0)),
            pl.BlockSpec((blk, dch), lambda i: (i, 0)),
            pl.BlockSpec((dch, f), full),
            pl.BlockSpec((f, f), full),
            pl.BlockSpec((1, f), full),
            pl.BlockSpec((f, f), full),
            pl.BlockSpec((1, f), full),
        ] + [pl.BlockSpec((1, f), full)] * 6,
        out_specs=pl.BlockSpec((blk, 128), lambda i: (i, 0)),
        out_shape=jax.ShapeDtypeStruct((n, 128), jnp.float32),
    )(a, g, x, wb, w1, vec(b1), w2, vec(b2),
      vec(s0), vec(t0), vec(s1), vec(t1), vec(s2), vec(t2))


def _edge_final_body(a2_ref, g2_ref, x1_ref, wb2_ref, s0_ref, t0_ref,
                     w3a_ref, w3b_ref, b3_ref, s3_ref, t3_ref,
                     out_ref, *, k):
    a2 = a2_ref[...]
    x1 = x1_ref[...]
    acc = None
    for s in range(k):
        e = g2_ref[s] - x1
        h = a2 + jnp.dot(e, wb2_ref[...], preferred_element_type=jnp.float32)
        h = jnp.maximum(h, 0.0) * s0_ref[...] + t0_ref[...]
        acc = h if acc is None else jnp.maximum(acc, h)
    o = (jnp.dot(x1, w3a_ref[...], preferred_element_type=jnp.float32)
         + jnp.dot(acc, w3b_ref[...], preferred_element_type=jnp.float32)
         + b3_ref[...])
    out_ref[...] = jnp.maximum(o, 0.0) * s3_ref[...] + t3_ref[...]


def _edge_final(a2, g2, x1, wb2, s0, t0, w3a, w3b, b3, s3, t3, blk=256):
    n, f2 = a2.shape
    dch = x1.shape[1]
    vec2 = lambda v: v.reshape(1, f2)
    full = lambda i: (0, 0)
    return pl.pallas_call(
        functools.partial(_edge_final_body, k=_K),
        grid=(n // blk,),
        in_specs=[
            pl.BlockSpec((blk, f2), lambda i: (i, 0)),
            pl.BlockSpec((_K, blk, dch), lambda i: (0, i, 0)),
            pl.BlockSpec((blk, dch), lambda i: (i, 0)),
            pl.BlockSpec((dch, f2), full),
            pl.BlockSpec((1, f2), full),
            pl.BlockSpec((1, f2), full),
            pl.BlockSpec((dch, f2), full),
            pl.BlockSpec((f2, f2), full),
            pl.BlockSpec((1, f2), full),
            pl.BlockSpec((1, f2), full),
            pl.BlockSpec((1, f2), full),
        ],
        out_specs=pl.BlockSpec((blk, f2), lambda i: (i, 0)),
        out_shape=jax.ShapeDtypeStruct((n, f2), jnp.float32),
    )(a2, g2, x1, wb2, vec2(s0), vec2(t0), w3a, w3b, vec2(b3), vec2(s3), vec2(t3))


def _idx_for_sc(idx_padded, n):
    flat = jnp.transpose(idx_padded[:, :_K]).reshape(-1)
    return flat.reshape(32, (_K * n) // (32 * 128), 128)


def _pad_rows(w, rows):
    return jnp.pad(w, ((0, rows - w.shape[0]), (0, 0)))


def kernel(cell_boxes, fusion_feat,
           W1_0, b1_0, g1_0, be1_0, W1_1, b1_1, g1_1, be1_1,
           W1_2, b1_2, g1_2, be1_2, W2_0, b2_0, g2_0, be2_0,
           W3_0, b3_0, g3_0, be3_0):
    del cell_boxes
    n, d = fusion_feat.shape
    inv = 1.0 / jnp.sqrt(jnp.float32(1.0 + _EPS))

    idx1, a1 = _knn_a(fusion_feat, W1_0[:d], b1_0)

    g1 = _sc_gather(fusion_feat, _idx_for_sc(idx1, n)).reshape(_K, n, -1)
    x1p = _edge_mlp3(a1, g1, fusion_feat, W1_0[d:], W1_1, b1_1, W1_2, b1_2,
                     g1_0 * inv, be1_0, g1_1 * inv, be1_1, g1_2 * inv, be1_2)

    f1 = W1_2.shape[1]
    idx2, a2 = _knn_a(x1p, _pad_rows(W2_0[:f1], 128), b2_0)

    g2 = _sc_gather(x1p, _idx_for_sc(idx2, n)).reshape(_K, n, -1)
    out = _edge_final(a2, g2, x1p, _pad_rows(W2_0[f1:], 128), g2_0 * inv, be2_0,
                      _pad_rows(W3_0[:f1], 128), W3_0[f1:], b3_0,
                      g3_0 * inv, be3_0)
    return out


# speedup vs baseline: 1.7241x; 1.0483x over previous
import functools

import jax
import jax.numpy as jnp
from jax import lax
from jax.experimental import pallas as pl
from jax.experimental.pallas import tpu as pltpu
from jax.experimental.pallas import tpu_sc as plsc

_EPS = 1e-5
_K = 20
_BIG = 3.0e38
_IDX_PAD = 32


def _topk_iter(vals, cols, sentinel, k):
    r = vals.shape[0]
    lane = lax.broadcasted_iota(jnp.int32, (r, _IDX_PAD), 1)
    idx_acc = jnp.zeros((r, _IDX_PAD), jnp.int32)
    m = jnp.min(vals, axis=1, keepdims=True)
    for j in range(k):
        c = jnp.min(jnp.where(vals == m, cols, sentinel), axis=1, keepdims=True)
        idx_acc = jnp.where(lane == j, c, idx_acc)
        if j < k - 1:
            vals = jnp.where(cols == c, _BIG, vals)
            m = jnp.min(vals, axis=1, keepdims=True)
    return idx_acc, m


def _knn_a_body(x_ref, xt_ref, wa_ref, b_ref, idx_ref, a_ref, *, n, k):
    x = x_ref[...]
    xt = xt_ref[...]
    sq_all = jnp.sum(xt * xt, axis=0, keepdims=True)
    sq_row = jnp.sum(x * x, axis=1, keepdims=True)
    d = sq_row + sq_all - 2.0 * jnp.dot(x, xt, preferred_element_type=jnp.float32)
    col = lax.broadcasted_iota(jnp.int32, d.shape, 1)
    r = d.shape[0]
    ng = n // 128
    nb = 5
    mv = [jnp.full((r, 128), _BIG, jnp.float32) for _ in range(nb)]
    av = [jnp.zeros((r, 128), jnp.int32) for _ in range(nb)]
    for g in range(ng):
        s = d[:, g * 128:(g + 1) * 128]
        p = [s < mv[i] for i in range(nb)]
        for i in range(nb - 1, 0, -1):
            mv[i] = jnp.where(p[i], jnp.where(p[i - 1], mv[i - 1], s), mv[i])
            av[i] = jnp.where(p[i], jnp.where(p[i - 1], av[i - 1], g), av[i])
        mv[0] = jnp.where(p[0], s, mv[0])
        av[0] = jnp.where(p[0], g, av[0])
    lane128 = lax.broadcasted_iota(jnp.int32, (r, 128), 1)
    vals = jnp.concatenate(mv, axis=1)
    cols = jnp.concatenate([a * 128 + lane128 for a in av], axis=1)
    idx_fast, t20 = _topk_iter(vals, cols, n, k)
    bad = jnp.max(jnp.where(mv[nb - 1] <= t20, 1, 0))
    idx_ref[...] = lax.cond(
        bad > 0,
        lambda: _topk_iter(d, col, n, k)[0],
        lambda: idx_fast)
    a_ref[...] = jnp.dot(x, wa_ref[...], preferred_element_type=jnp.float32) + b_ref[...]


def _knn_a(x, wa, b, blk=256):
    n, dch = x.shape
    f = wa.shape[1]
    xt = x.T
    return pl.pallas_call(
        functools.partial(_knn_a_body, n=n, k=_K),
        grid=(n // blk,),
        in_specs=[
            pl.BlockSpec((blk, dch), lambda i: (i, 0)),
            pl.BlockSpec((dch, n), lambda i: (0, 0)),
            pl.BlockSpec((dch, f), lambda i: (0, 0)),
            pl.BlockSpec((1, f), lambda i: (0, 0)),
        ],
        out_specs=[
            pl.BlockSpec((blk, _IDX_PAD), lambda i: (i, 0)),
            pl.BlockSpec((blk, f), lambda i: (i, 0)),
        ],
        out_shape=[
            jax.ShapeDtypeStruct((n, _IDX_PAD), jnp.int32),
            jax.ShapeDtypeStruct((n, f), jnp.float32),
        ],
    )(x, xt, wa, b.reshape(1, f))


def _sc_gather(table, idx, use_spmem=False):
    nw, nchunk, cw = idx.shape
    n, f = table.shape
    rows_out = nw * nchunk * cw
    per_w = nchunk * cw
    info = plsc.get_sparse_core_info()
    nc = info.num_cores
    mesh = plsc.VectorSubcoreMesh(core_axis_name="c", subcore_axis_name="s")

    @functools.partial(
        pl.kernel,
        out_type=jax.ShapeDtypeStruct((rows_out, f), jnp.float32),
        mesh=mesh,
        scratch_types=[
            pltpu.VMEM((nchunk, cw), jnp.int32),
            pltpu.VMEM((cw, f), jnp.float32),
            pltpu.VMEM((cw, f), jnp.float32),
        ] + ([pltpu.VMEM_SHARED((n, f), jnp.float32)] if use_spmem else [])
          + [pltpu.SemaphoreType.DMA, pltpu.SemaphoreType.DMA],
    )
    def gk(table_hbm, idx_hbm, out_hbm, idx_v, rows0, rows1, *rest):
        if use_spmem:
            shared, sem0, sem1 = rest
        else:
            sem0, sem1 = rest
        wid = lax.axis_index("s") * nc + lax.axis_index("c")
        base = wid * per_w
        if use_spmem:
            @pl.when(lax.axis_index("s") == 0)
            def _():
                pltpu.sync_copy(table_hbm, shared)

            plsc.subcore_barrier()
            src_tab = shared
        else:
            src_tab = table_hbm
        pltpu.sync_copy(idx_hbm.at[wid], idx_v)
        pltpu.async_copy(src_tab.at[idx_v.at[0]], rows0, sem0)

        def body(t, carry):
            j0 = 2 * t
            j1 = 2 * t + 1
            pltpu.async_copy(src_tab.at[idx_v.at[j1]], rows1, sem1)
            pltpu.make_async_copy(src_tab.at[idx_v.at[j0]], rows0, sem0).wait()
            pltpu.sync_copy(rows0, out_hbm.at[pl.ds(base + j0 * cw, cw)])

            @pl.when(t < nchunk // 2 - 1)
            def _():
                pltpu.async_copy(src_tab.at[idx_v.at[j1 + 1]], rows0, sem0)

            pltpu.make_async_copy(src_tab.at[idx_v.at[j1]], rows1, sem1).wait()
            pltpu.sync_copy(rows1, out_hbm.at[pl.ds(base + j1 * cw, cw)])
            return carry

        lax.fori_loop(0, nchunk // 2, body, 0)

    return gk(table, idx)


def _edge_mlp3_body(a_ref, g_ref, x_ref, wb_ref, w1_ref, b1_ref, w2_ref, b2_ref,
                    s0_ref, t0_ref, s1_ref, t1_ref, s2_ref, t2_ref,
                    *rest, k, pad):
    prev_ref = rest[0] if len(rest) == 2 else None
    x1_ref = rest[-1]
    a = a_ref[...]
    xi = x_ref[...]
    acc = None if prev_ref is None else prev_ref[...]
    for s in range(k):
        e = g_ref[s] - xi
        h = a + jnp.dot(e, wb_ref[...], preferred_element_type=jnp.float32)
        h = jnp.maximum(h, 0.0) * s0_ref[...] + t0_ref[...]
        h = jnp.maximum(
            jnp.dot(h, w1_ref[...], preferred_element_type=jnp.float32) + b1_ref[...],
            0.0) * s1_ref[...] + t1_ref[...]
        h = jnp.maximum(
            jnp.dot(h, w2_ref[...], preferred_element_type=jnp.float32) + b2_ref[...],
            0.0) * s2_ref[...] + t2_ref[...]
        acc = h if acc is None else jnp.maximum(acc, h)
    if pad:
        acc = jnp.concatenate(
            [acc, jnp.zeros((acc.shape[0], 128 - acc.shape[1]), acc.dtype)],
            axis=1)
    x1_ref[...] = acc


def _edge_mlp3(a, g, x, wb, w1, b1, w2, b2, s0, t0, s1, t1, s2, t2,
               prev=None, pad=False, blk=256):
    n, f = a.shape
    dch = x.shape[1]
    ks = g.shape[0]
    fo = 128 if pad else f
    vec = lambda v: v.reshape(1, f)
    full = lambda i: (0, 0)
    in_specs = [
        pl.BlockSpec((blk, f), lambda i: (i, 0)),
        pl.BlockSpec((ks, blk, dch), lambda i: (0, i, 0)),
        pl.BlockSpec((blk, dch), lambda i: (i, 0)),
        pl.BlockSpec((dch, f), full),
        pl.BlockSpec((f, f), full),
        pl.BlockSpec((1, f), full),
        pl.BlockSpec((f, f), full),
        pl.BlockSpec((1, f), full),
    ] + [pl.BlockSpec((1, f), full)] * 6
    args = [a, g, x, wb, w1, vec(b1), w2, vec(b2),
            vec(s0), vec(t0), vec(s1), vec(t1), vec(s2), vec(t2)]
    if prev is not None:
        in_specs.append(pl.BlockSpec((blk, f), lambda i: (i, 0)))
        args.append(prev)
    return pl.pallas_call(
        functools.partial(_edge_mlp3_body, k=ks, pad=pad),
        grid=(n // blk,),
        in_specs=in_specs,
        out_specs=pl.BlockSpec((blk, fo), lambda i: (i, 0)),
        out_shape=jax.ShapeDtypeStruct((n, fo), jnp.float32),
    )(*args)


def _edge_conv2_body(a2_ref, g2_ref, x1_ref, wb2_ref, s0_ref, t0_ref,
                     out_ref, *, k):
    a2 = a2_ref[...]
    x1 = x1_ref[...]
    acc = None
    for s in range(k):
        e = g2_ref[s] - x1
        h = a2 + jnp.dot(e, wb2_ref[...], preferred_element_type=jnp.float32)
        h = jnp.maximum(h, 0.0) * s0_ref[...] + t0_ref[...]
        acc = h if acc is None else jnp.maximum(acc, h)
    out_ref[...] = acc


def _edge_conv2(a2, g2, x1, wb2, s0, t0, blk=256):
    n, f2 = a2.shape
    dch = x1.shape[1]
    ks = g2.shape[0]
    vec2 = lambda v: v.reshape(1, f2)
    full = lambda i: (0, 0)
    return pl.pallas_call(
        functools.partial(_edge_conv2_body, k=ks),
        grid=(n // blk,),
        in_specs=[
            pl.BlockSpec((blk, f2), lambda i: (i, 0)),
            pl.BlockSpec((ks, blk, dch), lambda i: (0, i, 0)),
            pl.BlockSpec((blk, dch), lambda i: (i, 0)),
            pl.BlockSpec((dch, f2), full),
            pl.BlockSpec((1, f2), full),
            pl.BlockSpec((1, f2), full),
        ],
        out_specs=pl.BlockSpec((blk, f2), lambda i: (i, 0)),
        out_shape=jax.ShapeDtypeStruct((n, f2), jnp.float32),
    )(a2, g2, x1, wb2, vec2(s0), vec2(t0))


def _edge_final_body(a2_ref, g2_ref, x1_ref, wb2_ref, s0_ref, t0_ref,
                     w3a_ref, w3b_ref, b3_ref, s3_ref, t3_ref, prev_ref,
                     out_ref, *, k):
    a2 = a2_ref[...]
    x1 = x1_ref[...]
    acc = prev_ref[...]
    for s in range(k):
        e = g2_ref[s] - x1
        h = a2 + jnp.dot(e, wb2_ref[...], preferred_element_type=jnp.float32)
        h = jnp.maximum(h, 0.0) * s0_ref[...] + t0_ref[...]
        acc = jnp.maximum(acc, h)
    o = (jnp.dot(x1, w3a_ref[...], preferred_element_type=jnp.float32)
         + jnp.dot(acc, w3b_ref[...], preferred_element_type=jnp.float32)
         + b3_ref[...])
    out_ref[...] = jnp.maximum(o, 0.0) * s3_ref[...] + t3_ref[...]


def _edge_final(a2, g2, x1, wb2, s0, t0, w3a, w3b, b3, s3, t3, prev, blk=256):
    n, f2 = a2.shape
    dch = x1.shape[1]
    ks = g2.shape[0]
    vec2 = lambda v: v.reshape(1, f2)
    full = lambda i: (0, 0)
    return pl.pallas_call(
        functools.partial(_edge_final_body, k=ks),
        grid=(n // blk,),
        in_specs=[
            pl.BlockSpec((blk, f2), lambda i: (i, 0)),
            pl.BlockSpec((ks, blk, dch), lambda i: (0, i, 0)),
            pl.BlockSpec((blk, dch), lambda i: (i, 0)),
            pl.BlockSpec((dch, f2), full),
            pl.BlockSpec((1, f2), full),
            pl.BlockSpec((1, f2), full),
            pl.BlockSpec((dch, f2), full),
            pl.BlockSpec((f2, f2), full),
            pl.BlockSpec((1, f2), full),
            pl.BlockSpec((1, f2), full),
            pl.BlockSpec((1, f2), full),
            pl.BlockSpec((blk, f2), lambda i: (i, 0)),
        ],
        out_specs=pl.BlockSpec((blk, f2), lambda i: (i, 0)),
        out_shape=jax.ShapeDtypeStruct((n, f2), jnp.float32),
    )(a2, g2, x1, wb2, vec2(s0), vec2(t0), w3a, w3b, vec2(b3), vec2(s3),
      vec2(t3), prev)


def _idx_halves(idx_padded):
    flat = jnp.transpose(idx_padded[:, :_K]).reshape(-1)
    half = flat.shape[0] // 2
    return (flat[:half].reshape(32, -1, 128), flat[half:].reshape(32, -1, 128))


def _pad_rows(w, rows):
    return jnp.pad(w, ((0, rows - w.shape[0]), (0, 0)))


def kernel(cell_boxes, fusion_feat,
           W1_0, b1_0, g1_0, be1_0, W1_1, b1_1, g1_1, be1_1,
           W1_2, b1_2, g1_2, be1_2, W2_0, b2_0, g2_0, be2_0,
           W3_0, b3_0, g3_0, be3_0):
    del cell_boxes
    n, d = fusion_feat.shape
    inv = 1.0 / jnp.sqrt(jnp.float32(1.0 + _EPS))

    idx1, a1 = _knn_a(fusion_feat, W1_0[:d], b1_0)

    kh = _K // 2
    i1a, i1b = _idx_halves(idx1)
    g1a = _sc_gather(fusion_feat, i1a).reshape(kh, n, -1)
    g1b = _sc_gather(fusion_feat, i1b).reshape(kh, n, -1)
    mlp1 = (W1_0[d:], W1_1, b1_1, W1_2, b1_2,
            g1_0 * inv, be1_0, g1_1 * inv, be1_1, g1_2 * inv, be1_2)
    x1a = _edge_mlp3(a1, g1a, fusion_feat, *mlp1)
    x1p = _edge_mlp3(a1, g1b, fusion_feat, *mlp1, prev=x1a, pad=True)

    f1 = W1_2.shape[1]
    idx2, a2 = _knn_a(x1p, _pad_rows(W2_0[:f1], 128), b2_0)

    i2a, i2b = _idx_halves(idx2)
    g2a = _sc_gather(x1p, i2a, use_spmem=True).reshape(kh, n, -1)
    g2b = _sc_gather(x1p, i2b, use_spmem=True).reshape(kh, n, -1)
    wb2 = _pad_rows(W2_0[f1:], 128)
    x2a = _edge_conv2(a2, g2a, x1p, wb2, g2_0 * inv, be2_0)
    out = _edge_final(a2, g2b, x1p, wb2, g2_0 * inv, be2_0,
                      _pad_rows(W3_0[:f1], 128), W3_0[f1:], b3_0,
                      g3_0 * inv, be3_0, prev=x2a)
    return out


# speedup vs baseline: 1.8330x; 1.0632x over previous
import functools

import jax
import jax.numpy as jnp
from jax import lax
from jax.experimental import pallas as pl
from jax.experimental.pallas import tpu as pltpu
from jax.experimental.pallas import tpu_sc as plsc

_EPS = 1e-5
_K = 20
_BIG = 3.0e38
_IDX_PAD = 32


def _topk_iter(vals, cols, sentinel, k):
    r = vals.shape[0]
    lane = lax.broadcasted_iota(jnp.int32, (r, _IDX_PAD), 1)
    idx_acc = jnp.zeros((r, _IDX_PAD), jnp.int32)
    m = jnp.min(vals, axis=1, keepdims=True)
    for j in range(k):
        c = jnp.min(jnp.where(vals == m, cols, sentinel), axis=1, keepdims=True)
        idx_acc = jnp.where(lane == j, c, idx_acc)
        if j < k - 1:
            vals = jnp.where(cols == c, _BIG, vals)
            m = jnp.min(vals, axis=1, keepdims=True)
    return idx_acc, m


def _knn_a_body(x_ref, xt_ref, wa_ref, b_ref, idx_ref, a_ref, *, n, k):
    x = x_ref[...]
    xt = xt_ref[...]
    sq_all = jnp.sum(xt * xt, axis=0, keepdims=True)
    sq_row = jnp.sum(x * x, axis=1, keepdims=True)
    d = sq_row + sq_all - 2.0 * jnp.dot(x, xt, preferred_element_type=jnp.float32)
    col = lax.broadcasted_iota(jnp.int32, d.shape, 1)
    r = d.shape[0]
    ng = n // 128
    nb = 5
    mv = [jnp.full((r, 128), _BIG, jnp.float32) for _ in range(nb)]
    av = [jnp.zeros((r, 128), jnp.int32) for _ in range(nb)]
    for g in range(ng):
        s = d[:, g * 128:(g + 1) * 128]
        p = [s < mv[i] for i in range(nb)]
        for i in range(nb - 1, 0, -1):
            mv[i] = jnp.where(p[i], jnp.where(p[i - 1], mv[i - 1], s), mv[i])
            av[i] = jnp.where(p[i], jnp.where(p[i - 1], av[i - 1], g), av[i])
        mv[0] = jnp.where(p[0], s, mv[0])
        av[0] = jnp.where(p[0], g, av[0])
    lane128 = lax.broadcasted_iota(jnp.int32, (r, 128), 1)
    vals = jnp.concatenate(mv, axis=1)
    cols = jnp.concatenate([a * 128 + lane128 for a in av], axis=1)
    idx_fast, t20 = _topk_iter(vals, cols, n, k)
    bad = jnp.max(jnp.where(mv[nb - 1] <= t20, 1, 0))
    idx_ref[...] = lax.cond(
        bad > 0,
        lambda: _topk_iter(d, col, n, k)[0],
        lambda: idx_fast)
    a_ref[...] = jnp.dot(x, wa_ref[...], preferred_element_type=jnp.float32) + b_ref[...]


def _knn_a(x, wa, b, blk=512):
    n, dch = x.shape
    f = wa.shape[1]
    xt = x.T
    return pl.pallas_call(
        functools.partial(_knn_a_body, n=n, k=_K),
        grid=(n // blk,),
        in_specs=[
            pl.BlockSpec((blk, dch), lambda i: (i, 0)),
            pl.BlockSpec((dch, n), lambda i: (0, 0)),
            pl.BlockSpec((dch, f), lambda i: (0, 0)),
            pl.BlockSpec((1, f), lambda i: (0, 0)),
        ],
        out_specs=[
            pl.BlockSpec((blk, _IDX_PAD), lambda i: (i, 0)),
            pl.BlockSpec((blk, f), lambda i: (i, 0)),
        ],
        out_shape=[
            jax.ShapeDtypeStruct((n, _IDX_PAD), jnp.int32),
            jax.ShapeDtypeStruct((n, f), jnp.float32),
        ],
    )(x, xt, wa, b.reshape(1, f))


def _sc_gather(table, idx, use_spmem=False):
    nw, nchunk, cw = idx.shape
    n, f = table.shape
    rows_out = nw * nchunk * cw
    per_w = nchunk * cw
    info = plsc.get_sparse_core_info()
    nc = info.num_cores
    mesh = plsc.VectorSubcoreMesh(core_axis_name="c", subcore_axis_name="s")

    @functools.partial(
        pl.kernel,
        out_type=jax.ShapeDtypeStruct((rows_out, f), jnp.float32),
        mesh=mesh,
        scratch_types=[
            pltpu.VMEM((nchunk, cw), jnp.int32),
            pltpu.VMEM((cw, f), jnp.float32),
            pltpu.VMEM((cw, f), jnp.float32),
        ] + ([pltpu.VMEM_SHARED((n, f), jnp.float32)] if use_spmem else [])
          + [pltpu.SemaphoreType.DMA, pltpu.SemaphoreType.DMA],
    )
    def gk(table_hbm, idx_hbm, out_hbm, idx_v, rows0, rows1, *rest):
        if use_spmem:
            shared, sem0, sem1 = rest
        else:
            sem0, sem1 = rest
        wid = lax.axis_index("s") * nc + lax.axis_index("c")
        base = wid * per_w
        if use_spmem:
            @pl.when(lax.axis_index("s") == 0)
            def _():
                pltpu.sync_copy(table_hbm, shared)

            plsc.subcore_barrier()
            src_tab = shared
        else:
            src_tab = table_hbm
        pltpu.sync_copy(idx_hbm.at[wid], idx_v)
        pltpu.async_copy(src_tab.at[idx_v.at[0]], rows0, sem0)

        def body(t, carry):
            j0 = 2 * t
            j1 = 2 * t + 1
            pltpu.async_copy(src_tab.at[idx_v.at[j1]], rows1, sem1)
            pltpu.make_async_copy(src_tab.at[idx_v.at[j0]], rows0, sem0).wait()
            pltpu.sync_copy(rows0, out_hbm.at[pl.ds(base + j0 * cw, cw)])

            @pl.when(t < nchunk // 2 - 1)
            def _():
                pltpu.async_copy(src_tab.at[idx_v.at[j1 + 1]], rows0, sem0)

            pltpu.make_async_copy(src_tab.at[idx_v.at[j1]], rows1, sem1).wait()
            pltpu.sync_copy(rows1, out_hbm.at[pl.ds(base + j1 * cw, cw)])
            return carry

        lax.fori_loop(0, nchunk // 2, body, 0)

    return gk(table, idx)


def _edge_mlp3_body(a_ref, g_ref, x_ref, wb_ref, w1_ref, b1_ref, w2_ref, b2_ref,
                    s0_ref, t0_ref, s1_ref, t1_ref, s2_ref, t2_ref,
                    *rest, k, pad):
    prev_ref = rest[0] if len(rest) == 2 else None
    x1_ref = rest[-1]
    a = a_ref[...]
    xi = x_ref[...]
    acc = None if prev_ref is None else prev_ref[...]
    for s in range(k):
        e = g_ref[s] - xi
        h = a + jnp.dot(e, wb_ref[...], preferred_element_type=jnp.float32)
        h = jnp.maximum(h, 0.0) * s0_ref[...] + t0_ref[...]
        h = jnp.maximum(
            jnp.dot(h, w1_ref[...], preferred_element_type=jnp.float32) + b1_ref[...],
            0.0) * s1_ref[...] + t1_ref[...]
        h = jnp.maximum(
            jnp.dot(h, w2_ref[...], preferred_element_type=jnp.float32) + b2_ref[...],
            0.0) * s2_ref[...] + t2_ref[...]
        acc = h if acc is None else jnp.maximum(acc, h)
    if pad:
        acc = jnp.concatenate(
            [acc, jnp.zeros((acc.shape[0], 128 - acc.shape[1]), acc.dtype)],
            axis=1)
    x1_ref[...] = acc


def _edge_mlp3(a, g, x, wb, w1, b1, w2, b2, s0, t0, s1, t1, s2, t2,
               prev=None, pad=False, blk=256):
    n, f = a.shape
    dch = x.shape[1]
    ks = g.shape[0]
    fo = 128 if pad else f
    vec = lambda v: v.reshape(1, f)
    full = lambda i: (0, 0)
    in_specs = [
        pl.BlockSpec((blk, f), lambda i: (i, 0)),
        pl.BlockSpec((ks, blk, dch), lambda i: (0, i, 0)),
        pl.BlockSpec((blk, dch), lambda i: (i, 0)),
        pl.BlockSpec((dch, f), full),
        pl.BlockSpec((f, f), full),
        pl.BlockSpec((1, f), full),
        pl.BlockSpec((f, f), full),
        pl.BlockSpec((1, f), full),
    ] + [pl.BlockSpec((1, f), full)] * 6
    args = [a, g, x, wb, w1, vec(b1), w2, vec(b2),
            vec(s0), vec(t0), vec(s1), vec(t1), vec(s2), vec(t2)]
    if prev is not None:
        in_specs.append(pl.BlockSpec((blk, f), lambda i: (i, 0)))
        args.append(prev)
    return pl.pallas_call(
        functools.partial(_edge_mlp3_body, k=ks, pad=pad),
        grid=(n // blk,),
        in_specs=in_specs,
        out_specs=pl.BlockSpec((blk, fo), lambda i: (i, 0)),
        out_shape=jax.ShapeDtypeStruct((n, fo), jnp.float32),
    )(*args)


def _edge_conv2_body(a2_ref, g2_ref, x1_ref, wb2_ref, s0_ref, t0_ref,
                     out_ref, *, k):
    a2 = a2_ref[...]
    x1 = x1_ref[...]
    acc = None
    for s in range(k):
        e = g2_ref[s] - x1
        h = a2 + jnp.dot(e, wb2_ref[...], preferred_element_type=jnp.float32)
        h = jnp.maximum(h, 0.0) * s0_ref[...] + t0_ref[...]
        acc = h if acc is None else jnp.maximum(acc, h)
    out_ref[...] = acc


def _edge_conv2(a2, g2, x1, wb2, s0, t0, blk=256):
    n, f2 = a2.shape
    dch = x1.shape[1]
    ks = g2.shape[0]
    vec2 = lambda v: v.reshape(1, f2)
    full = lambda i: (0, 0)
    return pl.pallas_call(
        functools.partial(_edge_conv2_body, k=ks),
        grid=(n // blk,),
        in_specs=[
            pl.BlockSpec((blk, f2), lambda i: (i, 0)),
            pl.BlockSpec((ks, blk, dch), lambda i: (0, i, 0)),
            pl.BlockSpec((blk, dch), lambda i: (i, 0)),
            pl.BlockSpec((dch, f2), full),
            pl.BlockSpec((1, f2), full),
            pl.BlockSpec((1, f2), full),
        ],
        out_specs=pl.BlockSpec((blk, f2), lambda i: (i, 0)),
        out_shape=jax.ShapeDtypeStruct((n, f2), jnp.float32),
    )(a2, g2, x1, wb2, vec2(s0), vec2(t0))


def _edge_final_body(a2_ref, g2_ref, x1_ref, wb2_ref, s0_ref, t0_ref,
                     w3a_ref, w3b_ref, b3_ref, s3_ref, t3_ref, prev_ref,
                     out_ref, *, k):
    a2 = a2_ref[...]
    x1 = x1_ref[...]
    acc = prev_ref[...]
    for s in range(k):
        e = g2_ref[s] - x1
        h = a2 + jnp.dot(e, wb2_ref[...], preferred_element_type=jnp.float32)
        h = jnp.maximum(h, 0.0) * s0_ref[...] + t0_ref[...]
        acc = jnp.maximum(acc, h)
    o = (jnp.dot(x1, w3a_ref[...], preferred_element_type=jnp.float32)
         + jnp.dot(acc, w3b_ref[...], preferred_element_type=jnp.float32)
         + b3_ref[...])
    out_ref[...] = jnp.maximum(o, 0.0) * s3_ref[...] + t3_ref[...]


def _edge_final(a2, g2, x1, wb2, s0, t0, w3a, w3b, b3, s3, t3, prev, blk=256):
    n, f2 = a2.shape
    dch = x1.shape[1]
    ks = g2.shape[0]
    vec2 = lambda v: v.reshape(1, f2)
    full = lambda i: (0, 0)
    return pl.pallas_call(
        functools.partial(_edge_final_body, k=ks),
        grid=(n // blk,),
        in_specs=[
            pl.BlockSpec((blk, f2), lambda i: (i, 0)),
            pl.BlockSpec((ks, blk, dch), lambda i: (0, i, 0)),
            pl.BlockSpec((blk, dch), lambda i: (i, 0)),
            pl.BlockSpec((dch, f2), full),
            pl.BlockSpec((1, f2), full),
            pl.BlockSpec((1, f2), full),
            pl.BlockSpec((dch, f2), full),
            pl.BlockSpec((f2, f2), full),
            pl.BlockSpec((1, f2), full),
            pl.BlockSpec((1, f2), full),
            pl.BlockSpec((1, f2), full),
            pl.BlockSpec((blk, f2), lambda i: (i, 0)),
        ],
        out_specs=pl.BlockSpec((blk, f2), lambda i: (i, 0)),
        out_shape=jax.ShapeDtypeStruct((n, f2), jnp.float32),
    )(a2, g2, x1, wb2, vec2(s0), vec2(t0), w3a, w3b, vec2(b3), vec2(s3),
      vec2(t3), prev)


def _idx_halves(idx_padded):
    flat = jnp.transpose(idx_padded[:, :_K]).reshape(-1)
    half = flat.shape[0] // 2
    return (flat[:half].reshape(32, -1, 128), flat[half:].reshape(32, -1, 128))


def _pad_rows(w, rows):
    return jnp.pad(w, ((0, rows - w.shape[0]), (0, 0)))


def kernel(cell_boxes, fusion_feat,
           W1_0, b1_0, g1_0, be1_0, W1_1, b1_1, g1_1, be1_1,
           W1_2, b1_2, g1_2, be1_2, W2_0, b2_0, g2_0, be2_0,
           W3_0, b3_0, g3_0, be3_0):
    del cell_boxes
    n, d = fusion_feat.shape
    inv = 1.0 / jnp.sqrt(jnp.float32(1.0 + _EPS))

    idx1, a1 = _knn_a(fusion_feat, W1_0[:d], b1_0)

    kh = _K // 2
    i1a, i1b = _idx_halves(idx1)
    g1a = _sc_gather(fusion_feat, i1a).reshape(kh, n, -1)
    g1b = _sc_gather(fusion_feat, i1b).reshape(kh, n, -1)
    mlp1 = (W1_0[d:], W1_1, b1_1, W1_2, b1_2,
            g1_0 * inv, be1_0, g1_1 * inv, be1_1, g1_2 * inv, be1_2)
    x1a = _edge_mlp3(a1, g1a, fusion_feat, *mlp1)
    x1p = _edge_mlp3(a1, g1b, fusion_feat, *mlp1, prev=x1a, pad=True)

    f1 = W1_2.shape[1]
    idx2, a2 = _knn_a(x1p, _pad_rows(W2_0[:f1], 128), b2_0)

    i2a, i2b = _idx_halves(idx2)
    g2a = _sc_gather(x1p, i2a, use_spmem=True).reshape(kh, n, -1)
    g2b = _sc_gather(x1p, i2b, use_spmem=True).reshape(kh, n, -1)
    wb2 = _pad_rows(W2_0[f1:], 128)
    x2a = _edge_conv2(a2, g2a, x1p, wb2, g2_0 * inv, be2_0)
    out = _edge_final(a2, g2b, x1p, wb2, g2_0 * inv, be2_0,
                      _pad_rows(W3_0[:f1], 128), W3_0[f1:], b3_0,
                      g3_0 * inv, be3_0, prev=x2a)
    return out
